# baseline (speedup 1.0000x reference)
import functools

import jax
import jax.numpy as jnp
from jax.experimental import pallas as pl
from jax.experimental.pallas import tpu as pltpu
from jax.experimental.pallas import tpu_sc as plsc

NDIM = 3
JOINT = 16
HID = 64
NORM_FACTOR = 100.0
XPAD = 8

TR = 32
TC = 64
TH = TC


def _silu(v):
    return v * jax.nn.sigmoid(v)


def _gcl_kernel(slo_ref, snum_ref,
                hi_ref, xi_ref, mi_ref,
                hj_ref, xj_ref, mj_ref,
                w1a_ref, w1b_ref, w1d_ref, eb1_ref,
                ew2_ref, eb2_ref,
                xw1_ref, xb1_ref, xw2_ref,
                hw1a_ref, hw1b_ref, hb1_ref, hw2_ref, hb2_ref,
                osum_ref,
                hout_ref, xout_ref):
    rt = pl.program_id(0)
    h_i = hi_ref[...]
    x_i = xi_ref[...]
    m_i = mi_ref[...]

    w1b = w1b_ref[...]
    w1d = w1d_ref[...]
    ew2 = ew2_ref[...]
    eb2 = eb2_ref[...]
    xw1 = xw1_ref[...]
    xb1 = xb1_ref[...]
    xw2 = xw2_ref[...]
    osum = osum_ref[...]

    a_i = jnp.dot(h_i.astype(jnp.bfloat16), w1a_ref[...],
                  preferred_element_type=jnp.float32
                  ).astype(jnp.bfloat16) + eb1_ref[...]

    xi3 = x_i[:, None, :]
    mi3 = m_i[:, :, None]

    lo = slo_ref[rt]
    num = snum_ref[rt]

    def half(col):
        h_j = hj_ref[pl.ds(col, TH), :]
        x_j = xj_ref[pl.ds(col, TH), :]
        m_j = mj_ref[pl.ds(col, TH), :]

        b_j = jnp.dot(h_j.astype(jnp.bfloat16), w1b,
                      preferred_element_type=jnp.float32).astype(jnp.bfloat16)

        dxs = xi3 - x_j[None, :, :]
        d2 = jnp.sum(dxs * dxs, axis=2, keepdims=True)
        edge = mi3 == m_j[None, :, :]

        d2t = jnp.dot(d2.reshape(TR * TH, 1).astype(jnp.bfloat16), w1d,
                      preferred_element_type=jnp.float32
                      ).astype(jnp.bfloat16).reshape(TR, TH, HID)
        pre1 = a_i[:, None, :] + b_j[None, :, :] + d2t
        m1 = _silu(pre1).reshape(TR * TH, HID)
        m2 = _silu(jnp.dot(m1, ew2, preferred_element_type=jnp.float32
                           ).astype(jnp.bfloat16) + eb2)
        m2m = jnp.where(edge, m2.reshape(TR, TH, HID), jnp.bfloat16(0.0))
        agg_c = jnp.dot(osum, m2m.reshape(TR * TH, HID),
                        preferred_element_type=jnp.float32)

        sx = _silu(jnp.dot(m2, xw1, preferred_element_type=jnp.float32
                           ).astype(jnp.bfloat16) + xb1)
        phi = jnp.dot(sx, xw2,
                      preferred_element_type=jnp.float32).reshape(TR, TH, 1)
        wgt = jnp.where(edge, phi * jax.lax.rsqrt(d2 + 1e-8), 0.0)
        xd = (wgt * dxs).reshape(TR * TH, XPAD).astype(jnp.bfloat16)
        xacc_c = jnp.dot(osum, xd, preferred_element_type=jnp.float32)
        return agg_c, xacc_c

    def body(k, carry):
        agg, xacc = carry
        a1, x1 = half(lo + k * TC)
        return agg + a1, xacc + x1

    agg, xacc = jax.lax.fori_loop(
        0, num, body, (jnp.zeros((TR, HID), jnp.float32),
                       jnp.zeros((TR, XPAD), jnp.float32)))

    aggn = agg * (1.0 / NORM_FACTOR)
    pre_h = (jnp.dot(h_i, hw1a_ref[...], preferred_element_type=jnp.float32)
             + jnp.dot(aggn, hw1b_ref[...], preferred_element_type=jnp.float32)
             + hb1_ref[...])
    upd = jnp.dot(_silu(pre_h), hw2_ref[...],
                  preferred_element_type=jnp.float32) + hb2_ref[...]
    hout_ref[...] = h_i + upd
    xout_ref[...] = x_i + xacc * (1.0 / NORM_FACTOR)


def _gcl_layer(h_i, x_i, m_i2, h_j, x_j, m_j2, ct_lo, ct_num, n_pad_j, lp):
    n_pad_i = h_i.shape[0]
    nt_r = n_pad_i // TR

    bf16 = jnp.bfloat16
    w1a = lp['e_W1'][:HID].astype(bf16)
    w1b = lp['e_W1'][HID:2 * HID].astype(bf16)
    w1d = lp['e_W1'][2 * HID:2 * HID + 1].astype(bf16)
    eb1 = lp['e_b1'][None, :].astype(bf16)
    eb2 = lp['e_b2'][None, :].astype(bf16)
    xb1 = lp['x_b1'][None, :].astype(bf16)
    xw2 = lp['x_W2'].astype(bf16)
    ew2 = lp['e_W2'].astype(bf16)
    xw1 = lp['x_W1'].astype(bf16)
    osum = jnp.repeat(jnp.eye(TR, dtype=bf16), TH, axis=1)
    hw1a = lp['h_W1'][:HID]
    hw1b = lp['h_W1'][HID:]
    hb1 = lp['h_b1'][None, :]
    hb2 = lp['h_b2'][None, :]

    def im_row(rt, a, b):
        return (rt, 0)

    def im_full2(rt, a, b):
        return (0, 0)

    grid_spec = pltpu.PrefetchScalarGridSpec(
        num_scalar_prefetch=2,
        grid=(nt_r,),
        in_specs=[
            pl.BlockSpec((TR, HID), im_row),
            pl.BlockSpec((TR, XPAD), im_row),
            pl.BlockSpec((TR, 1), im_row),
            pl.BlockSpec((n_pad_j, HID), im_full2),
            pl.BlockSpec((n_pad_j, XPAD), im_full2),
            pl.BlockSpec((n_pad_j, 1), im_full2),
        ] + [pl.BlockSpec(w.shape, im_full2) for w in (
            w1a, w1b, w1d, eb1, ew2, eb2,
            xw1, xb1, xw2, hw1a, hw1b, hb1, lp['h_W2'], hb2, osum)],
        out_specs=[
            pl.BlockSpec((TR, HID), im_row),
            pl.BlockSpec((TR, XPAD), im_row),
        ],
    )
    h_new, x_new = pl.pallas_call(
        _gcl_kernel,
        grid_spec=grid_spec,
        out_shape=[
            jax.ShapeDtypeStruct((n_pad_i, HID), jnp.float32),
            jax.ShapeDtypeStruct((n_pad_i, XPAD), jnp.float32),
        ],
        compiler_params=pltpu.CompilerParams(
            dimension_semantics=("arbitrary",)),
    )(ct_lo, ct_num, h_i, x_i, m_i2, h_j, x_j, m_j2,
      w1a, w1b, w1d, eb1, ew2, eb2,
      xw1, xb1, xw2, hw1a, hw1b, hb1, lp['h_W2'], hb2, osum)
    return h_new, x_new


TRP = 512


def _enc_ln(xh, w1, b1, w2, b2):
    h = xh[:, NDIM:]
    e = jnp.dot(_silu(jnp.dot(h, w1, preferred_element_type=jnp.float32)
                      + b1), w2, preferred_element_type=jnp.float32) + b2
    mu = jnp.mean(e, axis=-1, keepdims=True)
    var = jnp.mean((e - mu) * (e - mu), axis=-1, keepdims=True)
    return (e - mu) / jnp.sqrt(var + 1e-5)


def _pre_lig_kernel(xh_ref, t_ref, aw1_ref, ab1_ref, aw2_ref, ab2_ref,
                    ew_ref, eb_ref, cw_ref, cb_ref, h1_ref, h2_ref):
    ln = _enc_ln(xh_ref[...], aw1_ref[...], ab1_ref[...],
                 aw2_ref[...], ab2_ref[...])
    ht = jnp.concatenate(
        [ln, jnp.broadcast_to(t_ref[...], (ln.shape[0], 1))], axis=1)
    h1_ref[...] = jnp.dot(ht, ew_ref[...],
                          preferred_element_type=jnp.float32) + eb_ref[...]
    h2_ref[...] = jnp.dot(ht, cw_ref[...],
                          preferred_element_type=jnp.float32) + cb_ref[...]


def _pre_ctx_kernel(xh_ref, rw1_ref, rb1_ref, rw2_ref, rb2_ref,
                    pw_ref, pb_ref, hp_ref):
    ln = _enc_ln(xh_ref[...], rw1_ref[...], rb1_ref[...],
                 rw2_ref[...], rb2_ref[...])
    hp_ref[...] = jnp.dot(ln, pw_ref[...],
                          preferred_element_type=jnp.float32) + pb_ref[...]


def _post_kernel(hll_ref, hlp_ref, xll_ref, xlp_ref, x0_ref,
                 eow_ref, eob_ref, cow_ref, cob_ref,
                 dw1_ref, db1_ref, dw2_ref, db2_ref, out_ref):
    hll = jnp.dot(hll_ref[...], eow_ref[...],
                  preferred_element_type=jnp.float32) + eob_ref[...]
    hlp = jnp.dot(hlp_ref[...], cow_ref[...],
                  preferred_element_type=jnp.float32) + cob_ref[...]
    hemb = (hll[:, :JOINT] + hlp[:, :JOINT]) * 0.5
    feat = jnp.dot(_silu(jnp.dot(hemb, dw1_ref[...],
                                 preferred_element_type=jnp.float32)
                         + db1_ref[...]), dw2_ref[...],
                   preferred_element_type=jnp.float32) + db2_ref[...]
    feat = jnp.nan_to_num(feat, nan=0.0, posinf=1.0, neginf=-1.0)
    x0 = x0_ref[...]
    vll = jnp.nan_to_num(xll_ref[...] - x0)
    vlp = jnp.nan_to_num(xlp_ref[...] - x0)
    vel = jnp.nan_to_num((vll + vlp) * 0.5, nan=0.0, posinf=1.0, neginf=-1.0)
    out_ref[...] = jnp.concatenate([vel[:, :NDIM], feat], axis=1)


def _rowwise_call(kfn, ins, out_shapes, n_pad):
    grid = (n_pad // TRP,)

    def spec(a):
        if a.shape[0] == n_pad:
            return pl.BlockSpec((TRP, a.shape[1]), lambda r: (r, 0))
        return pl.BlockSpec(a.shape, lambda r: (0, 0))

    return pl.pallas_call(
        kfn,
        grid=grid,
        in_specs=[spec(a) for a in ins],
        out_specs=[pl.BlockSpec((TRP, s[1]), lambda r: (r, 0))
                   for s in out_shapes],
        out_shape=[jax.ShapeDtypeStruct(s, jnp.float32) for s in out_shapes],
    )(*ins)


def _pad_rows(a, n_pad):
    return jnp.pad(a, ((0, n_pad - a.shape[0]), (0, 0)))


def _windows(mask_i, mask_j, n_pad_i, n_pad_j):
    n_i = mask_i.shape[0]
    n_j = mask_j.shape[0]
    nt_r = n_pad_i // TR
    ntp = ((nt_r + 15) // 16) * 16
    nbs = max(1, (n_j + 1).bit_length())
    tc_shift = TC.bit_length() - 1

    mesh = plsc.VectorSubcoreMesh(core_axis_name="c", subcore_axis_name="s")

    @functools.partial(
        pl.kernel, mesh=mesh,
        out_type=[jax.ShapeDtypeStruct((ntp,), jnp.int32),
                  jax.ShapeDtypeStruct((ntp,), jnp.int32)],
        scratch_types=[pltpu.VMEM((n_i,), jnp.int32),
                       pltpu.VMEM((n_j,), jnp.int32),
                       pltpu.VMEM((ntp,), jnp.int32),
                       pltpu.VMEM((ntp,), jnp.int32)],
        compiler_params=pltpu.CompilerParams(needs_layout_passes=False),
    )
    def route(mi_hbm, mj_hbm, lo_hbm, num_hbm, mi_v, mj_v, lo_v, num_v):
        wid = jax.lax.axis_index("s") * 2 + jax.lax.axis_index("c")

        @pl.when(wid == 0)
        def _():
            pltpu.sync_copy(mi_hbm, mi_v)
            pltpu.sync_copy(mj_hbm, mj_v)

            def chunk(ci, carry):
                base = ci * 16
                r0 = (base + jax.lax.iota(jnp.int32, 16)) * TR
                kmin = plsc.load_gather(mi_v, [jnp.minimum(r0, n_i - 1)])
                kmax = plsc.load_gather(
                    mi_v, [jnp.minimum(r0 + (TR - 1), n_i - 1)])

                def bsearch(key, is_left):
                    def step(_s, c):
                        lo, hi = c
                        mid = jnp.minimum((lo + hi) >> 1, n_j - 1)
                        v = plsc.load_gather(mj_v, [mid])
                        cond = (v < key) if is_left else (v <= key)
                        return (jnp.where(cond, mid + 1, lo),
                                jnp.where(cond, hi, mid))
                    z = jnp.zeros((16,), jnp.int32)
                    f = jnp.full((16,), n_j, jnp.int32)
                    return jax.lax.fori_loop(0, nbs, step, (z, f))[0]

                c_lo = bsearch(kmin, True)
                c_hi = bsearch(kmax, False)
                c0 = (c_lo >> 3) << 3
                nmb = (c_hi - c0 + (TC - 1)) >> tc_shift
                nmb = jnp.maximum(nmb, 0)
                nmb = jnp.where(r0 >= n_i, 0, nmb)
                lo_v[pl.ds(base, 16)] = c0
                num_v[pl.ds(base, 16)] = nmb
                return carry

            jax.lax.fori_loop(0, ntp // 16, chunk, 0)
            pltpu.sync_copy(lo_v, lo_hbm)
            pltpu.sync_copy(num_v, num_hbm)

    ct_lo, ct_num = route(mask_i, mask_j)
    return ct_lo[:nt_r], ct_num[:nt_r]


def kernel(xh_lig, xh_context, t, mask_lig, mask_context, params):
    n_l = xh_lig.shape[0]
    n_c = xh_context.shape[0]
    blk = 512
    n_pad_l = ((n_l + blk - 1) // blk) * blk
    n_pad_c = ((n_c + blk - 1) // blk) * blk
    p = params

    kj = jax.random.key(1234)
    x_l = xh_lig[:, :NDIM] + 1e-4 * jax.random.normal(
        kj, (n_l, NDIM), dtype=jnp.float32)

    xpad_l = _pad_rows(jnp.pad(x_l, ((0, 0), (0, XPAD - NDIM))), n_pad_l)
    xpad_p = _pad_rows(
        jnp.pad(xh_context[:, :NDIM], ((0, 0), (0, XPAD - NDIM))), n_pad_c)
    xh_l_pad = _pad_rows(xh_lig, n_pad_l)
    xh_c_pad = _pad_rows(xh_context, n_pad_c)
    mi_l = jnp.pad(mask_lig, (0, n_pad_l - n_l),
                   constant_values=1 << 20).reshape(n_pad_l, 1)
    mj_l2 = jnp.pad(mask_lig, (0, n_pad_l - n_l),
                    constant_values=1 << 21).reshape(n_pad_l, 1)
    mj_c2 = jnp.pad(mask_context, (0, n_pad_c - n_c),
                    constant_values=1 << 21).reshape(n_pad_c, 1)

    lo_ll, num_ll = _windows(mask_lig, mask_lig, n_pad_l, n_pad_l)
    lo_lp, num_lp = _windows(mask_lig, mask_context, n_pad_l, n_pad_c)

    ae = p['atom_enc']
    h_ll0, h_lp0 = _rowwise_call(
        _pre_lig_kernel,
        [xh_l_pad, t.reshape(1, 1), ae['W1'], ae['b1'][None], ae['W2'],
         ae['b2'][None], p['egnn_in_W'], p['egnn_in_b'][None],
         p['cross_in_l_W'], p['cross_in_l_b'][None]],
        [(n_pad_l, HID), (n_pad_l, HID)], n_pad_l)
    re = p['res_enc']
    hp0, = _rowwise_call(
        _pre_ctx_kernel,
        [xh_c_pad, re['W1'], re['b1'][None], re['W2'], re['b2'][None],
         p['cross_in_p_W'], p['cross_in_p_b'][None]],
        [(n_pad_c, HID)], n_pad_c)

    h, x = h_ll0, xpad_l
    for lp in p['egnn_layers']:
        h, x = _gcl_layer(h, x, mi_l, h, x, mj_l2, lo_ll, num_ll, n_pad_l, lp)
    h_ll, x_ll = h, x

    h, x = h_lp0, xpad_l
    for lp in p['cross_layers']:
        h, x = _gcl_layer(h, x, mi_l, hp0, xpad_p, mj_c2, lo_lp, num_lp,
                          n_pad_c, lp)
    h_lp, x_lp = h, x

    ad = p['atom_dec']
    outp, = _rowwise_call(
        _post_kernel,
        [h_ll, h_lp, x_ll, x_lp, xpad_l,
         p['egnn_out_W'], p['egnn_out_b'][None],
         p['cross_out_W'], p['cross_out_b'][None],
         ad['W1'], ad['b1'][None], ad['W2'], ad['b2'][None]],
        [(n_pad_l, xh_lig.shape[1])], n_pad_l)
    return outp[:n_l], jnp.zeros_like(xh_context)


# speedup vs baseline: 1.0862x; 1.0862x over previous
import functools

import jax
import jax.numpy as jnp
from jax.experimental import pallas as pl
from jax.experimental.pallas import tpu as pltpu
from jax.experimental.pallas import tpu_sc as plsc

NDIM = 3
JOINT = 16
HID = 64
NORM_FACTOR = 100.0
XPAD = 8

TR = 32
TC = 64
TH = TC


def _silu(v):
    return v * jax.nn.sigmoid(v)


def _gcl_kernel(slo_ref, snum_ref,
                hi_ref, xi_ref, mi_ref,
                hj_ref, xj_ref, mj_ref,
                w1a_ref, w1b_ref, w1d_ref, eb1_ref,
                ew2_ref, eb2_ref,
                xw1_ref, xb1_ref, xw2_ref,
                osum_ref,
                agg_ref, xacc_ref):
    rt = pl.program_id(0)
    h_i = hi_ref[...]
    x_i = xi_ref[...]
    m_i = mi_ref[...]

    w1b = w1b_ref[...]
    w1d = w1d_ref[...]
    ew2 = ew2_ref[...]
    eb2 = eb2_ref[...]
    xw1 = xw1_ref[...]
    xb1 = xb1_ref[...]
    xw2 = xw2_ref[...]
    osum = osum_ref[...]

    a_i = jnp.dot(h_i, w1a_ref[...],
                  preferred_element_type=jnp.float32) + eb1_ref[...]

    xi3 = x_i[:, None, :]
    mi3 = m_i[:, :, None]

    lo = slo_ref[rt]
    num = snum_ref[rt]

    def half(col):
        h_j = hj_ref[pl.ds(col, TH), :]
        x_j = xj_ref[pl.ds(col, TH), :]
        m_j = mj_ref[pl.ds(col, TH), :]

        b_j = jnp.dot(h_j, w1b, preferred_element_type=jnp.float32)

        dxs = xi3 - x_j[None, :, :]
        d2 = jnp.sum(dxs * dxs, axis=2, keepdims=True)
        edge = mi3 == m_j[None, :, :]

        d2t = jnp.dot(d2.reshape(TR * TH, 1), w1d,
                      preferred_element_type=jnp.float32).reshape(TR, TH, HID)
        pre1 = a_i[:, None, :] + b_j[None, :, :] + d2t
        m1 = _silu(pre1).reshape(TR * TH, HID)
        m2 = _silu(jnp.dot(m1, ew2, preferred_element_type=jnp.float32)
                   + eb2)
        m2m = jnp.where(edge, m2.reshape(TR, TH, HID), 0.0)
        agg_c = jnp.dot(osum, m2m.reshape(TR * TH, HID),
                        preferred_element_type=jnp.float32)

        sx = _silu(jnp.dot(m2, xw1, preferred_element_type=jnp.float32)
                   + xb1)
        phi = jnp.dot(sx, xw2,
                      preferred_element_type=jnp.float32).reshape(TR, TH, 1)
        wgt = jnp.where(edge, phi * jax.lax.rsqrt(d2 + 1e-8), 0.0)
        xacc_c = jnp.dot(osum, (wgt * dxs).reshape(TR * TH, XPAD),
                         preferred_element_type=jnp.float32)
        return agg_c, xacc_c

    def body(k, carry):
        agg, xacc = carry
        a1, x1 = half(lo + k * TC)
        return agg + a1, xacc + x1

    agg, xacc = jax.lax.fori_loop(
        0, num, body, (jnp.zeros((TR, HID), jnp.float32),
                       jnp.zeros((TR, XPAD), jnp.float32)))
    agg_ref[...] = agg
    xacc_ref[...] = xacc


def _gcl_layer(h_i, x_i, m_i2, h_j, x_j, m_j2, ct_lo, ct_num, n_pad_j, lp):
    n_pad_i = h_i.shape[0]
    nt_r = n_pad_i // TR

    w1a = lp['e_W1'][:HID]
    w1b = lp['e_W1'][HID:2 * HID]
    w1d = lp['e_W1'][2 * HID:2 * HID + 1]
    eb1 = lp['e_b1'][None, :]
    eb2 = lp['e_b2'][None, :]
    xb1 = lp['x_b1'][None, :]
    xw2 = lp['x_W2']
    ew2 = lp['e_W2']
    xw1 = lp['x_W1']
    osum = jnp.repeat(jnp.eye(TR, dtype=jnp.float32), TH, axis=1)

    def im_row(rt, a, b):
        return (rt, 0)

    def im_full2(rt, a, b):
        return (0, 0)

    grid_spec = pltpu.PrefetchScalarGridSpec(
        num_scalar_prefetch=2,
        grid=(nt_r,),
        in_specs=[
            pl.BlockSpec((TR, HID), im_row),
            pl.BlockSpec((TR, XPAD), im_row),
            pl.BlockSpec((TR, 1), im_row),
            pl.BlockSpec((n_pad_j, HID), im_full2),
            pl.BlockSpec((n_pad_j, XPAD), im_full2),
            pl.BlockSpec((n_pad_j, 1), im_full2),
        ] + [pl.BlockSpec(w.shape, im_full2) for w in (
            w1a, w1b, w1d, eb1, ew2, eb2,
            xw1, xb1, xw2, osum)],
        out_specs=[
            pl.BlockSpec((TR, HID), im_row),
            pl.BlockSpec((TR, XPAD), im_row),
        ],
    )
    agg, xacc = pl.pallas_call(
        _gcl_kernel,
        grid_spec=grid_spec,
        out_shape=[
            jax.ShapeDtypeStruct((n_pad_i, HID), jnp.float32),
            jax.ShapeDtypeStruct((n_pad_i, XPAD), jnp.float32),
        ],
        compiler_params=pltpu.CompilerParams(
            dimension_semantics=("arbitrary",)),
    )(ct_lo, ct_num, h_i, x_i, m_i2, h_j, x_j, m_j2,
      w1a, w1b, w1d, eb1, ew2, eb2,
      xw1, xb1, xw2, osum)
    return _rowwise_call(
        _upd_kernel,
        [h_i, x_i, agg, xacc, lp['h_W1'][:HID], lp['h_W1'][HID:],
         lp['h_b1'][None, :], lp['h_W2'], lp['h_b2'][None, :]],
        [(n_pad_i, HID), (n_pad_i, XPAD)], n_pad_i)


TRP = 512


def _upd_kernel(h_ref, x_ref, agg_ref, xacc_ref,
                hw1a_ref, hw1b_ref, hb1_ref, hw2_ref, hb2_ref,
                hout_ref, xout_ref):
    h = h_ref[...]
    aggn = agg_ref[...] * (1.0 / NORM_FACTOR)
    pre_h = (jnp.dot(h, hw1a_ref[...], preferred_element_type=jnp.float32)
             + jnp.dot(aggn, hw1b_ref[...], preferred_element_type=jnp.float32)
             + hb1_ref[...])
    hout_ref[...] = h + jnp.dot(_silu(pre_h), hw2_ref[...],
                                preferred_element_type=jnp.float32) + hb2_ref[...]
    xout_ref[...] = x_ref[...] + xacc_ref[...] * (1.0 / NORM_FACTOR)


def _enc_ln(xh, w1, b1, w2, b2):
    h = xh[:, NDIM:]
    e = jnp.dot(_silu(jnp.dot(h, w1, preferred_element_type=jnp.float32)
                      + b1), w2, preferred_element_type=jnp.float32) + b2
    mu = jnp.mean(e, axis=-1, keepdims=True)
    var = jnp.mean((e - mu) * (e - mu), axis=-1, keepdims=True)
    return (e - mu) / jnp.sqrt(var + 1e-5)


def _pre_lig_kernel(xh_ref, t_ref, aw1_ref, ab1_ref, aw2_ref, ab2_ref,
                    ew_ref, eb_ref, cw_ref, cb_ref, h1_ref, h2_ref):
    ln = _enc_ln(xh_ref[...], aw1_ref[...], ab1_ref[...],
                 aw2_ref[...], ab2_ref[...])
    ht = jnp.concatenate(
        [ln, jnp.broadcast_to(t_ref[...], (ln.shape[0], 1))], axis=1)
    h1_ref[...] = jnp.dot(ht, ew_ref[...],
                          preferred_element_type=jnp.float32) + eb_ref[...]
    h2_ref[...] = jnp.dot(ht, cw_ref[...],
                          preferred_element_type=jnp.float32) + cb_ref[...]


def _pre_ctx_kernel(xh_ref, rw1_ref, rb1_ref, rw2_ref, rb2_ref,
                    pw_ref, pb_ref, hp_ref):
    ln = _enc_ln(xh_ref[...], rw1_ref[...], rb1_ref[...],
                 rw2_ref[...], rb2_ref[...])
    hp_ref[...] = jnp.dot(ln, pw_ref[...],
                          preferred_element_type=jnp.float32) + pb_ref[...]


def _post_kernel(hll_ref, hlp_ref, xll_ref, xlp_ref, x0_ref,
                 eow_ref, eob_ref, cow_ref, cob_ref,
                 dw1_ref, db1_ref, dw2_ref, db2_ref, out_ref):
    hll = jnp.dot(hll_ref[...], eow_ref[...],
                  preferred_element_type=jnp.float32) + eob_ref[...]
    hlp = jnp.dot(hlp_ref[...], cow_ref[...],
                  preferred_element_type=jnp.float32) + cob_ref[...]
    hemb = (hll[:, :JOINT] + hlp[:, :JOINT]) * 0.5
    feat = jnp.dot(_silu(jnp.dot(hemb, dw1_ref[...],
                                 preferred_element_type=jnp.float32)
                         + db1_ref[...]), dw2_ref[...],
                   preferred_element_type=jnp.float32) + db2_ref[...]
    feat = jnp.nan_to_num(feat, nan=0.0, posinf=1.0, neginf=-1.0)
    x0 = x0_ref[...]
    vll = jnp.nan_to_num(xll_ref[...] - x0)
    vlp = jnp.nan_to_num(xlp_ref[...] - x0)
    vel = jnp.nan_to_num((vll + vlp) * 0.5, nan=0.0, posinf=1.0, neginf=-1.0)
    out_ref[...] = jnp.concatenate([vel[:, :NDIM], feat], axis=1)


def _rowwise_call(kfn, ins, out_shapes, n_pad):
    grid = (n_pad // TRP,)

    def spec(a):
        if a.shape[0] == n_pad:
            return pl.BlockSpec((TRP, a.shape[1]), lambda r: (r, 0))
        return pl.BlockSpec(a.shape, lambda r: (0, 0))

    return pl.pallas_call(
        kfn,
        grid=grid,
        in_specs=[spec(a) for a in ins],
        out_specs=[pl.BlockSpec((TRP, s[1]), lambda r: (r, 0))
                   for s in out_shapes],
        out_shape=[jax.ShapeDtypeStruct(s, jnp.float32) for s in out_shapes],
    )(*ins)


def _pad_rows(a, n_pad):
    return jnp.pad(a, ((0, n_pad - a.shape[0]), (0, 0)))


def _windows(mask_i, mask_j, n_pad_i, n_pad_j):
    n_i = mask_i.shape[0]
    n_j = mask_j.shape[0]
    nt_r = n_pad_i // TR
    ntp = ((nt_r + 15) // 16) * 16
    nbs = max(1, (n_j + 1).bit_length())
    tc_shift = TC.bit_length() - 1

    mesh = plsc.VectorSubcoreMesh(core_axis_name="c", subcore_axis_name="s")

    @functools.partial(
        pl.kernel, mesh=mesh,
        out_type=[jax.ShapeDtypeStruct((ntp,), jnp.int32),
                  jax.ShapeDtypeStruct((ntp,), jnp.int32)],
        scratch_types=[pltpu.VMEM((n_i,), jnp.int32),
                       pltpu.VMEM((n_j,), jnp.int32),
                       pltpu.VMEM((ntp,), jnp.int32),
                       pltpu.VMEM((ntp,), jnp.int32)],
        compiler_params=pltpu.CompilerParams(needs_layout_passes=False),
    )
    def route(mi_hbm, mj_hbm, lo_hbm, num_hbm, mi_v, mj_v, lo_v, num_v):
        wid = jax.lax.axis_index("s") * 2 + jax.lax.axis_index("c")

        @pl.when(wid == 0)
        def _():
            pltpu.sync_copy(mi_hbm, mi_v)
            pltpu.sync_copy(mj_hbm, mj_v)

            def chunk(ci, carry):
                base = ci * 16
                r0 = (base + jax.lax.iota(jnp.int32, 16)) * TR
                kmin = plsc.load_gather(mi_v, [jnp.minimum(r0, n_i - 1)])
                kmax = plsc.load_gather(
                    mi_v, [jnp.minimum(r0 + (TR - 1), n_i - 1)])

                def bsearch(key, is_left):
                    def step(_s, c):
                        lo, hi = c
                        mid = jnp.minimum((lo + hi) >> 1, n_j - 1)
                        v = plsc.load_gather(mj_v, [mid])
                        cond = (v < key) if is_left else (v <= key)
                        return (jnp.where(cond, mid + 1, lo),
                                jnp.where(cond, hi, mid))
                    z = jnp.zeros((16,), jnp.int32)
                    f = jnp.full((16,), n_j, jnp.int32)
                    return jax.lax.fori_loop(0, nbs, step, (z, f))[0]

                c_lo = bsearch(kmin, True)
                c_hi = bsearch(kmax, False)
                c0 = (c_lo >> 3) << 3
                nmb = (c_hi - c0 + (TC - 1)) >> tc_shift
                nmb = jnp.maximum(nmb, 0)
                nmb = jnp.where(r0 >= n_i, 0, nmb)
                lo_v[pl.ds(base, 16)] = c0
                num_v[pl.ds(base, 16)] = nmb
                return carry

            jax.lax.fori_loop(0, ntp // 16, chunk, 0)
            pltpu.sync_copy(lo_v, lo_hbm)
            pltpu.sync_copy(num_v, num_hbm)

    ct_lo, ct_num = route(mask_i, mask_j)
    return ct_lo[:nt_r], ct_num[:nt_r]


def kernel(xh_lig, xh_context, t, mask_lig, mask_context, params):
    n_l = xh_lig.shape[0]
    n_c = xh_context.shape[0]
    blk = 512
    n_pad_l = ((n_l + blk - 1) // blk) * blk
    n_pad_c = ((n_c + blk - 1) // blk) * blk
    p = params

    kj = jax.random.key(1234)
    x_l = xh_lig[:, :NDIM] + 1e-4 * jax.random.normal(
        kj, (n_l, NDIM), dtype=jnp.float32)

    xpad_l = _pad_rows(jnp.pad(x_l, ((0, 0), (0, XPAD - NDIM))), n_pad_l)
    xpad_p = _pad_rows(
        jnp.pad(xh_context[:, :NDIM], ((0, 0), (0, XPAD - NDIM))), n_pad_c)
    xh_l_pad = _pad_rows(xh_lig, n_pad_l)
    xh_c_pad = _pad_rows(xh_context, n_pad_c)
    mi_l = jnp.pad(mask_lig, (0, n_pad_l - n_l),
                   constant_values=1 << 20).reshape(n_pad_l, 1)
    mj_l2 = jnp.pad(mask_lig, (0, n_pad_l - n_l),
                    constant_values=1 << 21).reshape(n_pad_l, 1)
    mj_c2 = jnp.pad(mask_context, (0, n_pad_c - n_c),
                    constant_values=1 << 21).reshape(n_pad_c, 1)

    lo_ll, num_ll = _windows(mask_lig, mask_lig, n_pad_l, n_pad_l)
    lo_lp, num_lp = _windows(mask_lig, mask_context, n_pad_l, n_pad_c)

    ae = p['atom_enc']
    h_ll0, h_lp0 = _rowwise_call(
        _pre_lig_kernel,
        [xh_l_pad, t.reshape(1, 1), ae['W1'], ae['b1'][None], ae['W2'],
         ae['b2'][None], p['egnn_in_W'], p['egnn_in_b'][None],
         p['cross_in_l_W'], p['cross_in_l_b'][None]],
        [(n_pad_l, HID), (n_pad_l, HID)], n_pad_l)
    re = p['res_enc']
    hp0, = _rowwise_call(
        _pre_ctx_kernel,
        [xh_c_pad, re['W1'], re['b1'][None], re['W2'], re['b2'][None],
         p['cross_in_p_W'], p['cross_in_p_b'][None]],
        [(n_pad_c, HID)], n_pad_c)

    h, x = h_ll0, xpad_l
    for lp in p['egnn_layers']:
        h, x = _gcl_layer(h, x, mi_l, h, x, mj_l2, lo_ll, num_ll, n_pad_l, lp)
    h_ll, x_ll = h, x

    h, x = h_lp0, xpad_l
    for lp in p['cross_layers']:
        h, x = _gcl_layer(h, x, mi_l, hp0, xpad_p, mj_c2, lo_lp, num_lp,
                          n_pad_c, lp)
    h_lp, x_lp = h, x

    ad = p['atom_dec']
    outp, = _rowwise_call(
        _post_kernel,
        [h_ll, h_lp, x_ll, x_lp, xpad_l,
         p['egnn_out_W'], p['egnn_out_b'][None],
         p['cross_out_W'], p['cross_out_b'][None],
         ad['W1'], ad['b1'][None], ad['W2'], ad['b2'][None]],
        [(n_pad_l, xh_lig.shape[1])], n_pad_l)
    return outp[:n_l], jnp.zeros_like(xh_context)
